# baseline (speedup 1.0000x reference)
import functools

import jax
import jax.numpy as jnp
from jax import lax
from jax.experimental import pallas as pl
from jax.experimental.pallas import tpu as pltpu
from jax.experimental.pallas import tpu_sc as plsc

N = 10000
E = 320000
D = 128
NG = 64
NC = 2
NS = 16
NW = NC * NS
EW = E // NW
CH = 80
NCH = EW // CH
BE = 4000
EH = E // 2
CHH = 40
NBH = EH // BE

_SELU_L = 1.0507009873554804934193349852946
_SELU_A = 1.6732632423543772848170429916717


def _selu(z):
    return _SELU_L * jnp.where(z > 0, z, _SELU_A * (jnp.exp(z) - 1.0))


def _mesh():
    return plsc.VectorSubcoreMesh(core_axis_name="c", subcore_axis_name="s")


@functools.cache
def _build_sc_gather(n_e, CH):
    NCH = n_e // NW // CH
    TB = (CH, 2 * D)
    PB = (CH, D)

    @functools.partial(
        pl.kernel,
        out_type=(jax.ShapeDtypeStruct((n_e, 2 * D), jnp.float32),
                  jax.ShapeDtypeStruct((n_e, D), jnp.float32)),
        mesh=_mesh(),
        scratch_types=[
            pltpu.VMEM((NCH, CH), jnp.int32),
            pltpu.VMEM((NCH, CH), jnp.int32),
            pltpu.VMEM(TB, jnp.float32),
            pltpu.VMEM(TB, jnp.float32),
            pltpu.VMEM(PB, jnp.float32),
            pltpu.VMEM(PB, jnp.float32),
            pltpu.SemaphoreType.DMA,
            pltpu.SemaphoreType.DMA,
            pltpu.SemaphoreType.DMA,
            pltpu.SemaphoreType.DMA,
        ],
    )
    def k(t_hbm, p_hbm, row_hbm, col_hbm, gr_hbm, gc_hbm,
          rbuf, cbuf, tbuf0, tbuf1, pbuf0, pbuf1, gsem0, gsem1, wsem0, wsem1):
        c = lax.axis_index("c")
        s = lax.axis_index("s")
        wid = s * NC + c
        pltpu.sync_copy(row_hbm.at[wid], rbuf)
        pltpu.sync_copy(col_hbm.at[wid], cbuf)

        def gath(j, tb, pb, gsem):
            pltpu.async_copy(t_hbm.at[rbuf.at[j]], tb, gsem)
            pltpu.async_copy(p_hbm.at[cbuf.at[j]], pb, gsem)

        def wait_g(tb, pb, gsem):
            pltpu.make_async_copy(t_hbm.at[rbuf.at[0]], tb, gsem).wait()
            pltpu.make_async_copy(p_hbm.at[cbuf.at[0]], pb, gsem).wait()

        def wait_w(tb, pb, wsem):
            pltpu.make_async_copy(tb, gr_hbm.at[pl.ds(0, CH)], wsem).wait()
            pltpu.make_async_copy(pb, gc_hbm.at[pl.ds(0, CH)], wsem).wait()

        def do_iter(j, tb, pb, gsem, wsem, tbo, pbo, gsemo, wsemo):
            @pl.when(j + 1 < NCH)
            def _():
                @pl.when(j >= 1)
                def _():
                    wait_w(tbo, pbo, wsemo)
                gath(j + 1, tbo, pbo, gsemo)
            wait_g(tb, pb, gsem)
            e0 = (wid * NCH + j) * CH
            pltpu.async_copy(tb, gr_hbm.at[pl.ds(e0, CH)], wsem)
            pltpu.async_copy(pb, gc_hbm.at[pl.ds(e0, CH)], wsem)

        gath(0, tbuf0, pbuf0, gsem0)

        def body(j, carry):
            @pl.when(j % 2 == 0)
            def _():
                do_iter(j, tbuf0, pbuf0, gsem0, wsem0,
                        tbuf1, pbuf1, gsem1, wsem1)

            @pl.when(j % 2 == 1)
            def _():
                do_iter(j, tbuf1, pbuf1, gsem1, wsem1,
                        tbuf0, pbuf0, gsem0, wsem0)
            return carry

        lax.fori_loop(0, NCH, body, 0)
        wait_w(tbuf0, pbuf0, wsem0)
        wait_w(tbuf1, pbuf1, wsem1)

    return k


@functools.cache
def _build_sc_scatter(ds_, read_vals, n_e, CH):
    NCH = n_e // NW // CH
    scratch = [
        pltpu.VMEM((NCH, CH), jnp.int32),
        pltpu.VMEM((CH, ds_), jnp.float32),
        pltpu.VMEM((CH, ds_), jnp.float32),
        pltpu.VMEM_SHARED((N, ds_), jnp.float32),
        pltpu.SemaphoreType.DMA,
        pltpu.SemaphoreType.DMA,
    ]

    def body(vals_hbm, col_hbm, zeros_hbm, out_hbm, cbuf, vbuf0, vbuf1, acc,
             lsem0, lsem1):
        c = lax.axis_index("c")
        s = lax.axis_index("s")
        wid = s * NC + c

        @pl.when(s == 0)
        def _():
            pltpu.sync_copy(zeros_hbm, acc)

        if not read_vals:
            def fill(i, carry):
                vbuf0[i] = jnp.full((ds_,), 1.0, jnp.float32)
                return carry
            lax.fori_loop(0, CH, fill, 0)

        plsc.subcore_barrier()
        pltpu.sync_copy(col_hbm.at[wid], cbuf)

        if read_vals:
            def load(j, vb, lsem):
                e0 = (wid * NCH + j) * CH
                pltpu.async_copy(vals_hbm.at[pl.ds(e0, CH)], vb, lsem)

            def wait_l(vb, lsem):
                pltpu.make_async_copy(vals_hbm.at[pl.ds(0, CH)], vb, lsem
                                      ).wait()

            def do_iter(j, vb, lsem, vbo, lsemo):
                @pl.when(j + 1 < NCH)
                def _():
                    load(j + 1, vbo, lsemo)
                wait_l(vb, lsem)
                pltpu.sync_copy(vb, acc.at[cbuf.at[j]], add=True)

            load(0, vbuf0, lsem0)

            def chunk(j, carry):
                @pl.when(j % 2 == 0)
                def _():
                    do_iter(j, vbuf0, lsem0, vbuf1, lsem1)

                @pl.when(j % 2 == 1)
                def _():
                    do_iter(j, vbuf1, lsem1, vbuf0, lsem0)
                return carry
        else:
            def chunk(j, carry):
                pltpu.sync_copy(vbuf0, acc.at[cbuf.at[j]], add=True)
                return carry

        lax.fori_loop(0, NCH, chunk, 0)
        plsc.subcore_barrier()

        @pl.when(s == 0)
        def _():
            pltpu.sync_copy(acc, out_hbm.at[c])

    if read_vals:
        def k(vals_hbm, col_hbm, zeros_hbm, out_hbm, cbuf, vbuf0, vbuf1, acc,
              lsem0, lsem1):
            body(vals_hbm, col_hbm, zeros_hbm, out_hbm, cbuf, vbuf0, vbuf1,
                 acc, lsem0, lsem1)
    else:
        def k(col_hbm, zeros_hbm, out_hbm, cbuf, vbuf0, vbuf1, acc, lsem0,
              lsem1):
            body(None, col_hbm, zeros_hbm, out_hbm, cbuf, vbuf0, vbuf1, acc,
                 lsem0, lsem1)

    return pl.kernel(
        k,
        out_type=jax.ShapeDtypeStruct((NC, N, ds_), jnp.float32),
        mesh=_mesh(),
        scratch_types=scratch,
    )


def _dot(a, b):
    return jnp.dot(a, b, preferred_element_type=jnp.float32)


@functools.cache
def _build_prep(dea):
    def body(x, u, b2, we1, wn1, be1, bn1, t_out, p_out):
        onehot = (b2[...] == lax.broadcasted_iota(jnp.int32, (1, NG), 1)
                  ).astype(jnp.float32)
        uw = _dot(u[...], we1[2 * D + dea:3 * D + dea, :])
        q = _dot(x[...], we1[0:D, :]) + _dot(onehot, uw) + be1[...]
        pn = _dot(x[...], wn1[0:D, :]) + bn1[...]
        t_out[...] = jnp.concatenate([q, pn], axis=1)
        p_out[...] = _dot(x[...], we1[D:2 * D, :])

    return pl.pallas_call(
        body,
        out_shape=(jax.ShapeDtypeStruct((N, 2 * D), jnp.float32),
                   jax.ShapeDtypeStruct((N, D), jnp.float32)),
    )


def _affine_step0(st_refs, g_ref, bb_ref, scr_ref):
    @pl.when(pl.program_id(0) == 0)
    def _():
        ssum = sum(jnp.sum(r[...], axis=0) for r in st_refs)
        mean = ssum[0:1] * (1.0 / E)
        var = ssum[1:2] * (1.0 / E) - mean * mean
        a = g_ref[...] * lax.rsqrt(var + 1e-5)
        scr_ref[...] = jnp.concatenate([a, bb_ref[...] - mean * a], axis=0)


def _pre_apply(x, scr_ref):
    return _selu(x * scr_ref[0:1] + scr_ref[1:2])


@functools.cache
def _build_pass(d_x, w_row0, npst, adds_cols, stats, n_e, nblk):
    nadds = len(adds_cols)

    def body(*refs):
        i = 0
        x_ref = refs[i]; i += 1
        w_ref = refs[i]; i += 1
        b_ref = refs[i]; i += 1
        pst_refs = refs[i:i + npst]; i += npst
        if npst:
            g_ref = refs[i]; i += 1
            bb_ref = refs[i]; i += 1
        add_refs = refs[i:i + nadds]; i += nadds
        z_ref = refs[i]; i += 1
        st_ref = refs[i] if stats else None
        scr_ref = refs[-1]

        xb = x_ref[...]
        if npst:
            _affine_step0(pst_refs, g_ref, bb_ref, scr_ref)
            xb = _pre_apply(xb, scr_ref)
        z = _dot(xb, w_ref[w_row0:w_row0 + d_x, :]) + b_ref[...]
        for ad in add_refs:
            z = z + ad[...]
        z_ref[...] = z
        if stats:
            st_ref[...] = jnp.stack(
                [jnp.sum(z, axis=0), jnp.sum(z * z, axis=0)])[None]

    def wspec(nrows):
        return pl.BlockSpec((nrows, D), lambda i: (0, 0))

    in_specs = [
        pl.BlockSpec((BE, d_x), lambda i: (i, 0)),
        wspec(w_row0 + d_x),
        wspec(1),
    ]
    in_specs += [pl.BlockSpec((nblk, 2, D), lambda i: (0, 0, 0))] * npst
    if npst:
        in_specs += [wspec(1), wspec(1)]
    for cb in adds_cols:
        in_specs.append(pl.BlockSpec((BE, D), lambda i, cb=cb: (i, cb)))
    out_shape = [jax.ShapeDtypeStruct((n_e, D), jnp.float32)]
    out_specs = [pl.BlockSpec((BE, D), lambda i: (i, 0))]
    if stats:
        out_shape.append(jax.ShapeDtypeStruct((nblk, 2, D), jnp.float32))
        out_specs.append(pl.BlockSpec((1, 2, D), lambda i: (i, 0, 0)))

    return pl.pallas_call(
        body,
        grid=(n_e // BE,),
        in_specs=in_specs,
        out_specs=out_specs,
        out_shape=out_shape,
        scratch_shapes=[pltpu.VMEM((2, D), jnp.float32)],
    )


@functools.cache
def _build_pass_e3(npst, n_e, nblk):
    def body(*refs):
        i = 0
        z2 = refs[i]; i += 1
        pst_refs = refs[i:i + npst]; i += npst
        g2, bb2, w3, b3, wn, g = refs[i:i + 6]; i += 6
        eo_ref, nz_ref, st_ref, scr = refs[i:i + 4]
        _affine_step0(pst_refs, g2, bb2, scr)
        a2 = _pre_apply(z2[...], scr)
        eo = _dot(a2, w3[...]) + b3[...]
        eo_ref[...] = eo
        nz = _dot(eo, wn[D:2 * D, :]) + g[...]
        nz_ref[...] = nz
        st_ref[...] = jnp.stack(
            [jnp.sum(nz, axis=0), jnp.sum(nz * nz, axis=0)])[None]

    return pl.pallas_call(
        body,
        grid=(n_e // BE,),
        in_specs=[
            pl.BlockSpec((BE, D), lambda i: (i, 0)),
        ] + [pl.BlockSpec((nblk, 2, D), lambda i: (0, 0, 0))] * npst + [
            pl.BlockSpec((1, D), lambda i: (0, 0)),
            pl.BlockSpec((1, D), lambda i: (0, 0)),
            pl.BlockSpec((D, D), lambda i: (0, 0)),
            pl.BlockSpec((1, D), lambda i: (0, 0)),
            pl.BlockSpec((2 * D, D), lambda i: (0, 0)),
            pl.BlockSpec((BE, D), lambda i: (i, 1)),
        ],
        out_specs=[
            pl.BlockSpec((BE, D), lambda i: (i, 0)),
            pl.BlockSpec((BE, D), lambda i: (i, 0)),
            pl.BlockSpec((1, 2, D), lambda i: (i, 0, 0)),
        ],
        out_shape=[
            jax.ShapeDtypeStruct((n_e, D), jnp.float32),
            jax.ShapeDtypeStruct((n_e, D), jnp.float32),
            jax.ShapeDtypeStruct((nblk, 2, D), jnp.float32),
        ],
        scratch_shapes=[pltpu.VMEM((2, D), jnp.float32)],
    )


def _bn(z, g, bb):
    mean = jnp.mean(z, axis=0, keepdims=True)
    var = jnp.var(z, axis=0, keepdims=True)
    return (z - mean) * lax.rsqrt(var + 1e-5) * g + bb


@functools.cache
def _build_node_fused():
    def body(x, acca, accb, cinv, u, b2,
             wm1, bm1, gm1, gmb1, wm2, bm2, gm2, gmb2, wm3, bm3,
             wg1, bg1, gg1, ggb1, wg2, bg2, gg2, ggb2, wg3, bg3,
             xn_ref, un_ref):
        agg = (acca[0] + acca[1] + accb[0] + accb[1]) * cinv[...]
        onehot = (b2[...] == lax.broadcasted_iota(jnp.int32, (1, NG), 1)
                  ).astype(jnp.float32)
        ub = _dot(onehot, _dot(u[...], wm1[2 * D:3 * D, :]))
        z = (_dot(x[...], wm1[0:D, :]) + _dot(agg, wm1[D:2 * D, :])
             + ub + bm1[...])
        h1 = _selu(_bn(z, gm1[...], gmb1[...]))
        z2 = _dot(h1, wm2[...]) + bm2[...]
        h2 = _selu(_bn(z2, gm2[...], gmb2[...]))
        xn = _dot(h2, wm3[...]) + bm3[...]
        xn_ref[...] = xn

        nc = jnp.sum(onehot, axis=0)
        nmean = lax.dot_general(onehot, xn, (((0,), (0,)), ((), ()))
                                ) / jnp.maximum(nc, 1.0)[:, None]
        gz = (_dot(u[...], wg1[0:D, :]) + _dot(nmean, wg1[D:2 * D, :])
              + bg1[...])
        d1 = _selu(_bn(gz, gg1[...], ggb1[...]))
        gz2 = _dot(d1, wg2[...]) + bg2[...]
        d2 = _selu(_bn(gz2, gg2[...], ggb2[...]))
        un_ref[...] = _dot(d2, wg3[...]) + bg3[...]

    return pl.pallas_call(
        body,
        out_shape=(jax.ShapeDtypeStruct((N, D), jnp.float32),
                   jax.ShapeDtypeStruct((NG, D), jnp.float32)),
    )


def _r(v):
    return v.reshape(1, D)


def kernel(x, edge_attr, u, params, edge_index, batch):
    row = edge_index[0]
    col = edge_index[1]
    col2_full = col.reshape(NW, E // NW // CH, CH)
    row2 = [row[h * EH:(h + 1) * EH].reshape(NW, EH // NW // CHH, CHH)
            for h in range(2)]
    col2 = [col[h * EH:(h + 1) * EH].reshape(NW, EH // NW // CHH, CHH)
            for h in range(2)]
    batch2 = batch.reshape(N, 1)
    zeros128 = jnp.zeros((N, D), jnp.float32)

    counts = _build_sc_scatter(D, False, E, CH)(col2_full, zeros128)
    c = counts[0, :, 0] + counts[1, :, 0]
    cinv = (1.0 / jnp.maximum(c, 1.0)).reshape(N, 1)

    ea = [edge_attr[0:EH], edge_attr[EH:]]
    H = range(2)
    for lp in params:
        (We1, be1, g1, bb1), (We2, be2, g2, bb2), (We3, be3) = lp['edge']
        (Wn1, bn1, gn1, bnb1), (Wn2, bn2, gn2, bnb2), (Wn3, bn3) = lp['node1']
        (Wm1, bm1, gm1, gmb1), (Wm2, bm2, gm2, gmb2), (Wm3, bm3) = lp['node2']
        (Wg1, bg1, gg1, ggb1), (Wg2, bg2, gg2, ggb2), (Wg3, bg3) = lp['global']
        dea = We1.shape[0] - (2 * D + D)

        T, P = _build_prep(dea)(
            x, u, batch2, We1, Wn1, _r(be1), _r(bn1))
        G = [_build_sc_gather(EH, CHH)(T, P, row2[h], col2[h]) for h in H]

        zb = jnp.zeros((1, D), jnp.float32)
        e1 = [_build_pass(dea, 2 * D, 0, (0, 0), True, EH, NBH)(
            ea[h], We1, zb, G[h][0], G[h][1]) for h in H]
        st1 = [e1[h][1] for h in H]
        e2 = [_build_pass(D, 0, 2, (), True, EH, NBH)(
            e1[h][0], We2, _r(be2), st1[0], st1[1], _r(g1), _r(bb1))
            for h in H]
        st2 = [e2[h][1] for h in H]
        e3 = [_build_pass_e3(2, EH, NBH)(
            e2[h][0], st2[0], st2[1], _r(g2), _r(bb2), We3, _r(be3), Wn1,
            G[h][0]) for h in H]
        st3 = [e3[h][2] for h in H]
        n2 = [_build_pass(D, 0, 2, (), True, EH, NBH)(
            e3[h][1], Wn2, _r(bn2), st3[0], st3[1], _r(gn1), _r(bnb1))
            for h in H]
        st4 = [n2[h][1] for h in H]
        hh = [_build_pass(D, 0, 2, (), False, EH, NBH)(
            n2[h][0], Wn3, _r(bn3), st4[0], st4[1], _r(gn2), _r(bnb2))[0]
            for h in H]

        acc = [_build_sc_scatter(D, True, EH, CHH)(hh[h], col2[h], zeros128)
               for h in H]

        x, u = _build_node_fused()(
            x, acc[0], acc[1], cinv, u, batch2,
            Wm1, _r(bm1), _r(gm1), _r(gmb1),
            Wm2, _r(bm2), _r(gm2), _r(gmb2), Wm3, _r(bm3),
            Wg1, _r(bg1), _r(gg1), _r(ggb1),
            Wg2, _r(bg2), _r(gg2), _r(ggb2), Wg3, _r(bg3))
        ea = [e3[h][0] for h in H]
    return u


# speedup vs baseline: 1.0320x; 1.0320x over previous
import functools

import jax
import jax.numpy as jnp
from jax import lax
from jax.experimental import pallas as pl
from jax.experimental.pallas import tpu as pltpu
from jax.experimental.pallas import tpu_sc as plsc

N = 10000
E = 320000
D = 128
NG = 64
NC = 2
NS = 16
NW = NC * NS
EW = E // NW
CH = 80
NCH = EW // CH
BE = 4000
EH = E // 2
CHH = 40
NBH = EH // BE

_SELU_L = 1.0507009873554804934193349852946
_SELU_A = 1.6732632423543772848170429916717


def _selu(z):
    return _SELU_L * jnp.where(z > 0, z, _SELU_A * (jnp.exp(z) - 1.0))


def _mesh():
    return plsc.VectorSubcoreMesh(core_axis_name="c", subcore_axis_name="s")


@functools.cache
def _build_sc_gather(n_e, CH):
    NCH = n_e // NW // CH
    TB = (CH, 2 * D)
    PB = (CH, D)

    @functools.partial(
        pl.kernel,
        out_type=(jax.ShapeDtypeStruct((n_e, 2 * D), jnp.float32),
                  jax.ShapeDtypeStruct((n_e, D), jnp.float32)),
        mesh=_mesh(),
        scratch_types=[
            pltpu.VMEM((NCH, CH), jnp.int32),
            pltpu.VMEM((NCH, CH), jnp.int32),
            pltpu.VMEM(TB, jnp.float32),
            pltpu.VMEM(TB, jnp.float32),
            pltpu.VMEM(PB, jnp.float32),
            pltpu.VMEM(PB, jnp.float32),
            pltpu.SemaphoreType.DMA,
            pltpu.SemaphoreType.DMA,
            pltpu.SemaphoreType.DMA,
            pltpu.SemaphoreType.DMA,
        ],
    )
    def k(t_hbm, p_hbm, row_hbm, col_hbm, gr_hbm, gc_hbm,
          rbuf, cbuf, tbuf0, tbuf1, pbuf0, pbuf1, gsem0, gsem1, wsem0, wsem1):
        c = lax.axis_index("c")
        s = lax.axis_index("s")
        wid = s * NC + c
        pltpu.sync_copy(row_hbm.at[wid], rbuf)
        pltpu.sync_copy(col_hbm.at[wid], cbuf)

        def gath(j, tb, pb, gsem):
            pltpu.async_copy(t_hbm.at[rbuf.at[j]], tb, gsem)
            pltpu.async_copy(p_hbm.at[cbuf.at[j]], pb, gsem)

        def wait_g(tb, pb, gsem):
            pltpu.make_async_copy(t_hbm.at[rbuf.at[0]], tb, gsem).wait()
            pltpu.make_async_copy(p_hbm.at[cbuf.at[0]], pb, gsem).wait()

        def wait_w(tb, pb, wsem):
            pltpu.make_async_copy(tb, gr_hbm.at[pl.ds(0, CH)], wsem).wait()
            pltpu.make_async_copy(pb, gc_hbm.at[pl.ds(0, CH)], wsem).wait()

        def do_iter(j, tb, pb, gsem, wsem, tbo, pbo, gsemo, wsemo):
            @pl.when(j + 1 < NCH)
            def _():
                @pl.when(j >= 1)
                def _():
                    wait_w(tbo, pbo, wsemo)
                gath(j + 1, tbo, pbo, gsemo)
            wait_g(tb, pb, gsem)
            e0 = (wid * NCH + j) * CH
            pltpu.async_copy(tb, gr_hbm.at[pl.ds(e0, CH)], wsem)
            pltpu.async_copy(pb, gc_hbm.at[pl.ds(e0, CH)], wsem)

        gath(0, tbuf0, pbuf0, gsem0)

        def body(j, carry):
            @pl.when(j % 2 == 0)
            def _():
                do_iter(j, tbuf0, pbuf0, gsem0, wsem0,
                        tbuf1, pbuf1, gsem1, wsem1)

            @pl.when(j % 2 == 1)
            def _():
                do_iter(j, tbuf1, pbuf1, gsem1, wsem1,
                        tbuf0, pbuf0, gsem0, wsem0)
            return carry

        lax.fori_loop(0, NCH, body, 0)
        wait_w(tbuf0, pbuf0, wsem0)
        wait_w(tbuf1, pbuf1, wsem1)

    return k


@functools.cache
def _build_sc_scatter(ds_, read_vals, n_e, CH):
    NCH = n_e // NW // CH
    scratch = [
        pltpu.VMEM((NCH, CH), jnp.int32),
        pltpu.VMEM((CH, ds_), jnp.float32),
        pltpu.VMEM((CH, ds_), jnp.float32),
        pltpu.VMEM_SHARED((N, ds_), jnp.float32),
        pltpu.SemaphoreType.DMA,
        pltpu.SemaphoreType.DMA,
    ]

    def body(vals_hbm, col_hbm, zeros_hbm, out_hbm, cbuf, vbuf0, vbuf1, acc,
             lsem0, lsem1):
        c = lax.axis_index("c")
        s = lax.axis_index("s")
        wid = s * NC + c

        @pl.when(s == 0)
        def _():
            pltpu.sync_copy(zeros_hbm, acc)

        if not read_vals:
            def fill(i, carry):
                vbuf0[i] = jnp.full((ds_,), 1.0, jnp.float32)
                return carry
            lax.fori_loop(0, CH, fill, 0)

        plsc.subcore_barrier()
        pltpu.sync_copy(col_hbm.at[wid], cbuf)

        if read_vals:
            def load(j, vb, lsem):
                e0 = (wid * NCH + j) * CH
                pltpu.async_copy(vals_hbm.at[pl.ds(e0, CH)], vb, lsem)

            def wait_l(vb, lsem):
                pltpu.make_async_copy(vals_hbm.at[pl.ds(0, CH)], vb, lsem
                                      ).wait()

            def do_iter(j, vb, lsem, vbo, lsemo):
                @pl.when(j + 1 < NCH)
                def _():
                    load(j + 1, vbo, lsemo)
                wait_l(vb, lsem)
                pltpu.sync_copy(vb, acc.at[cbuf.at[j]], add=True)

            load(0, vbuf0, lsem0)

            def chunk(j, carry):
                @pl.when(j % 2 == 0)
                def _():
                    do_iter(j, vbuf0, lsem0, vbuf1, lsem1)

                @pl.when(j % 2 == 1)
                def _():
                    do_iter(j, vbuf1, lsem1, vbuf0, lsem0)
                return carry
        else:
            def chunk(j, carry):
                pltpu.sync_copy(vbuf0, acc.at[cbuf.at[j]], add=True)
                return carry

        lax.fori_loop(0, NCH, chunk, 0)
        plsc.subcore_barrier()

        @pl.when(s == 0)
        def _():
            pltpu.sync_copy(acc, out_hbm.at[c])

    if read_vals:
        def k(vals_hbm, col_hbm, zeros_hbm, out_hbm, cbuf, vbuf0, vbuf1, acc,
              lsem0, lsem1):
            body(vals_hbm, col_hbm, zeros_hbm, out_hbm, cbuf, vbuf0, vbuf1,
                 acc, lsem0, lsem1)
    else:
        def k(col_hbm, zeros_hbm, out_hbm, cbuf, vbuf0, vbuf1, acc, lsem0,
              lsem1):
            body(None, col_hbm, zeros_hbm, out_hbm, cbuf, vbuf0, vbuf1, acc,
                 lsem0, lsem1)

    return pl.kernel(
        k,
        out_type=jax.ShapeDtypeStruct((NC, N, ds_), jnp.float32),
        mesh=_mesh(),
        scratch_types=scratch,
    )


def _dot(a, b):
    return jnp.dot(a, b, preferred_element_type=jnp.float32)


@functools.cache
def _build_prep(dea):
    def body(x, u, b2, we1, wn1, be1, bn1, t_out, p_out):
        onehot = (b2[...] == lax.broadcasted_iota(jnp.int32, (1, NG), 1)
                  ).astype(jnp.float32)
        uw = _dot(u[...], we1[2 * D + dea:3 * D + dea, :])
        q = _dot(x[...], we1[0:D, :]) + _dot(onehot, uw) + be1[...]
        pn = _dot(x[...], wn1[0:D, :]) + bn1[...]
        t_out[...] = jnp.concatenate([q, pn], axis=1)
        p_out[...] = _dot(x[...], we1[D:2 * D, :])

    return pl.pallas_call(
        body,
        out_shape=(jax.ShapeDtypeStruct((N, 2 * D), jnp.float32),
                   jax.ShapeDtypeStruct((N, D), jnp.float32)),
    )


def _affine_step0(st_refs, g_ref, bb_ref, scr_ref):
    @pl.when(pl.program_id(0) == 0)
    def _():
        ssum = sum(jnp.sum(r[...], axis=0) for r in st_refs)
        mean = ssum[0:1] * (1.0 / E)
        var = ssum[1:2] * (1.0 / E) - mean * mean
        a = g_ref[...] * lax.rsqrt(var + 1e-5)
        scr_ref[...] = jnp.concatenate([a, bb_ref[...] - mean * a], axis=0)


def _pre_apply(x, scr_ref):
    return _selu(x * scr_ref[0:1] + scr_ref[1:2])


@functools.cache
def _build_pass(d_x, w_row0, pst_nblks, adds_cols, stats, n_e, nblk):
    npst = len(pst_nblks)
    nadds = len(adds_cols)

    def body(*refs):
        i = 0
        x_ref = refs[i]; i += 1
        w_ref = refs[i]; i += 1
        b_ref = refs[i]; i += 1
        pst_refs = refs[i:i + npst]; i += npst
        if npst:
            g_ref = refs[i]; i += 1
            bb_ref = refs[i]; i += 1
        add_refs = refs[i:i + nadds]; i += nadds
        z_ref = refs[i]; i += 1
        st_ref = refs[i] if stats else None
        scr_ref = refs[-1]

        xb = x_ref[...]
        if npst:
            _affine_step0(pst_refs, g_ref, bb_ref, scr_ref)
            xb = _pre_apply(xb, scr_ref)
        z = _dot(xb, w_ref[w_row0:w_row0 + d_x, :]) + b_ref[...]
        for ad in add_refs:
            z = z + ad[...]
        z_ref[...] = z
        if stats:
            st_ref[...] = jnp.stack(
                [jnp.sum(z, axis=0), jnp.sum(z * z, axis=0)])[None]

    def wspec(nrows):
        return pl.BlockSpec((nrows, D), lambda i: (0, 0))

    in_specs = [
        pl.BlockSpec((BE, d_x), lambda i: (i, 0)),
        wspec(w_row0 + d_x),
        wspec(1),
    ]
    in_specs += [pl.BlockSpec((nbk, 2, D), lambda i: (0, 0, 0))
                 for nbk in pst_nblks]
    if npst:
        in_specs += [wspec(1), wspec(1)]
    for cb in adds_cols:
        in_specs.append(pl.BlockSpec((BE, D), lambda i, cb=cb: (i, cb)))
    out_shape = [jax.ShapeDtypeStruct((n_e, D), jnp.float32)]
    out_specs = [pl.BlockSpec((BE, D), lambda i: (i, 0))]
    if stats:
        out_shape.append(jax.ShapeDtypeStruct((nblk, 2, D), jnp.float32))
        out_specs.append(pl.BlockSpec((1, 2, D), lambda i: (i, 0, 0)))

    return pl.pallas_call(
        body,
        grid=(n_e // BE,),
        in_specs=in_specs,
        out_specs=out_specs,
        out_shape=out_shape,
        scratch_shapes=[pltpu.VMEM((2, D), jnp.float32)],
    )


@functools.cache
def _build_pass_e3(pst_nblks, n_e, nblk):
    npst = len(pst_nblks)
    def body(*refs):
        i = 0
        z2 = refs[i]; i += 1
        pst_refs = refs[i:i + npst]; i += npst
        g2, bb2, w3, b3, wn, g = refs[i:i + 6]; i += 6
        eo_ref, nz_ref, st_ref, scr = refs[i:i + 4]
        _affine_step0(pst_refs, g2, bb2, scr)
        a2 = _pre_apply(z2[...], scr)
        eo = _dot(a2, w3[...]) + b3[...]
        eo_ref[...] = eo
        nz = _dot(eo, wn[D:2 * D, :]) + g[...]
        nz_ref[...] = nz
        st_ref[...] = jnp.stack(
            [jnp.sum(nz, axis=0), jnp.sum(nz * nz, axis=0)])[None]

    return pl.pallas_call(
        body,
        grid=(n_e // BE,),
        in_specs=[
            pl.BlockSpec((BE, D), lambda i: (i, 0)),
        ] + [pl.BlockSpec((nbk, 2, D), lambda i: (0, 0, 0))
             for nbk in pst_nblks] + [
            pl.BlockSpec((1, D), lambda i: (0, 0)),
            pl.BlockSpec((1, D), lambda i: (0, 0)),
            pl.BlockSpec((D, D), lambda i: (0, 0)),
            pl.BlockSpec((1, D), lambda i: (0, 0)),
            pl.BlockSpec((2 * D, D), lambda i: (0, 0)),
            pl.BlockSpec((BE, D), lambda i: (i, 1)),
        ],
        out_specs=[
            pl.BlockSpec((BE, D), lambda i: (i, 0)),
            pl.BlockSpec((BE, D), lambda i: (i, 0)),
            pl.BlockSpec((1, 2, D), lambda i: (i, 0, 0)),
        ],
        out_shape=[
            jax.ShapeDtypeStruct((n_e, D), jnp.float32),
            jax.ShapeDtypeStruct((n_e, D), jnp.float32),
            jax.ShapeDtypeStruct((nblk, 2, D), jnp.float32),
        ],
        scratch_shapes=[pltpu.VMEM((2, D), jnp.float32)],
    )


def _bn(z, g, bb):
    mean = jnp.mean(z, axis=0, keepdims=True)
    var = jnp.var(z, axis=0, keepdims=True)
    return (z - mean) * lax.rsqrt(var + 1e-5) * g + bb


@functools.cache
def _build_node_fused(nseg):
    def body(*refs):
        x = refs[0]
        accs = refs[1:1 + nseg]
        (cinv, u, b2,
         wm1, bm1, gm1, gmb1, wm2, bm2, gm2, gmb2, wm3, bm3,
         wg1, bg1, gg1, ggb1, wg2, bg2, gg2, ggb2, wg3, bg3,
         xn_ref, un_ref) = refs[1 + nseg:]
        agg = sum(a[0] + a[1] for a in accs) * cinv[...]
        onehot = (b2[...] == lax.broadcasted_iota(jnp.int32, (1, NG), 1)
                  ).astype(jnp.float32)
        ub = _dot(onehot, _dot(u[...], wm1[2 * D:3 * D, :]))
        z = (_dot(x[...], wm1[0:D, :]) + _dot(agg, wm1[D:2 * D, :])
             + ub + bm1[...])
        h1 = _selu(_bn(z, gm1[...], gmb1[...]))
        z2 = _dot(h1, wm2[...]) + bm2[...]
        h2 = _selu(_bn(z2, gm2[...], gmb2[...]))
        xn = _dot(h2, wm3[...]) + bm3[...]
        xn_ref[...] = xn

        nc = jnp.sum(onehot, axis=0)
        nmean = lax.dot_general(onehot, xn, (((0,), (0,)), ((), ()))
                                ) / jnp.maximum(nc, 1.0)[:, None]
        gz = (_dot(u[...], wg1[0:D, :]) + _dot(nmean, wg1[D:2 * D, :])
              + bg1[...])
        d1 = _selu(_bn(gz, gg1[...], ggb1[...]))
        gz2 = _dot(d1, wg2[...]) + bg2[...]
        d2 = _selu(_bn(gz2, gg2[...], ggb2[...]))
        un_ref[...] = _dot(d2, wg3[...]) + bg3[...]

    return pl.pallas_call(
        body,
        out_shape=(jax.ShapeDtypeStruct((N, D), jnp.float32),
                   jax.ShapeDtypeStruct((NG, D), jnp.float32)),
    )


def _r(v):
    return v.reshape(1, D)


SPLITS = ((0, E),)


def kernel(x, edge_attr, u, params, edge_index, batch):
    row = edge_index[0]
    col = edge_index[1]
    col2_full = col.reshape(NW, E // NW // CH, CH)
    row2 = [row[s:s + n].reshape(NW, n // NW // CH, CH) for s, n in SPLITS]
    col2 = [col[s:s + n].reshape(NW, n // NW // CH, CH) for s, n in SPLITS]
    batch2 = batch.reshape(N, 1)
    zeros128 = jnp.zeros((N, D), jnp.float32)

    counts = _build_sc_scatter(D, False, E, CH)(col2_full, zeros128)
    c = counts[0, :, 0] + counts[1, :, 0]
    cinv = (1.0 / jnp.maximum(c, 1.0)).reshape(N, 1)

    nseg = len(SPLITS)
    ea = [edge_attr[s:s + n] if nseg > 1 else edge_attr for s, n in SPLITS]
    H = range(nseg)
    for lp in params:
        (We1, be1, g1, bb1), (We2, be2, g2, bb2), (We3, be3) = lp['edge']
        (Wn1, bn1, gn1, bnb1), (Wn2, bn2, gn2, bnb2), (Wn3, bn3) = lp['node1']
        (Wm1, bm1, gm1, gmb1), (Wm2, bm2, gm2, gmb2), (Wm3, bm3) = lp['node2']
        (Wg1, bg1, gg1, ggb1), (Wg2, bg2, gg2, ggb2), (Wg3, bg3) = lp['global']
        dea = We1.shape[0] - (2 * D + D)

        T, P = _build_prep(dea)(
            x, u, batch2, We1, Wn1, _r(be1), _r(bn1))
        G = [_build_sc_gather(SPLITS[h][1], CH)(T, P, row2[h], col2[h])
             for h in H]

        zb = jnp.zeros((1, D), jnp.float32)

        def nb(h):
            return SPLITS[h][1] // BE

        nbs = tuple(nb(h) for h in H)
        e1 = [_build_pass(dea, 2 * D, (), (0, 0), True, SPLITS[h][1], nb(h))(
            ea[h], We1, zb, G[h][0], G[h][1]) for h in H]
        st1 = [e1[h][1] for h in H]
        e2 = [_build_pass(D, 0, nbs, (), True, SPLITS[h][1], nb(h))(
            e1[h][0], We2, _r(be2), *st1, _r(g1), _r(bb1))
            for h in H]
        st2 = [e2[h][1] for h in H]
        e3 = [_build_pass_e3(nbs, SPLITS[h][1], nb(h))(
            e2[h][0], *st2, _r(g2), _r(bb2), We3, _r(be3), Wn1,
            G[h][0]) for h in H]
        st3 = [e3[h][2] for h in H]
        n2 = [_build_pass(D, 0, nbs, (), True, SPLITS[h][1], nb(h))(
            e3[h][1], Wn2, _r(bn2), *st3, _r(gn1), _r(bnb1))
            for h in H]
        st4 = [n2[h][1] for h in H]
        hh = [_build_pass(D, 0, nbs, (), False, SPLITS[h][1], nb(h))(
            n2[h][0], Wn3, _r(bn3), *st4, _r(gn2), _r(bnb2))[0]
            for h in H]

        acc = [_build_sc_scatter(D, True, SPLITS[h][1], CH)(
            hh[h], col2[h], zeros128) for h in H]

        x, u = _build_node_fused(nseg)(
            x, *acc, cinv, u, batch2,
            Wm1, _r(bm1), _r(gm1), _r(gmb1),
            Wm2, _r(bm2), _r(gm2), _r(gmb2), Wm3, _r(bm3),
            Wg1, _r(bg1), _r(gg1), _r(ggb1),
            Wg2, _r(bg2), _r(gg2), _r(ggb2), Wg3, _r(bg3))
        ea = [e3[h][0] for h in H]
    return u


# speedup vs baseline: 1.0337x; 1.0017x over previous
import functools

import jax
import jax.numpy as jnp
from jax import lax
from jax.experimental import pallas as pl
from jax.experimental.pallas import tpu as pltpu
from jax.experimental.pallas import tpu_sc as plsc

N = 10000
E = 320000
D = 128
NG = 64
NC = 2
NS = 16
NW = NC * NS
EW = E // NW
CH = 80
NCH = EW // CH
BE = 4000
EH = E // 2
CHH = 40
NBH = EH // BE

_SELU_L = 1.0507009873554804934193349852946
_SELU_A = 1.6732632423543772848170429916717


def _selu(z):
    return _SELU_L * jnp.where(z > 0, z, _SELU_A * (jnp.exp(z) - 1.0))


def _mesh():
    return plsc.VectorSubcoreMesh(core_axis_name="c", subcore_axis_name="s")


@functools.cache
def _build_sc_gather(n_e, CH):
    NCH = n_e // NW // CH
    TB = (CH, 2 * D)
    PB = (CH, D)

    @functools.partial(
        pl.kernel,
        out_type=(jax.ShapeDtypeStruct((n_e, 2 * D), jnp.float32),
                  jax.ShapeDtypeStruct((n_e, D), jnp.float32)),
        mesh=_mesh(),
        scratch_types=[
            pltpu.VMEM((NCH, CH), jnp.int32),
            pltpu.VMEM((NCH, CH), jnp.int32),
            pltpu.VMEM(TB, jnp.float32),
            pltpu.VMEM(TB, jnp.float32),
            pltpu.VMEM(PB, jnp.float32),
            pltpu.VMEM(PB, jnp.float32),
            pltpu.SemaphoreType.DMA,
            pltpu.SemaphoreType.DMA,
            pltpu.SemaphoreType.DMA,
            pltpu.SemaphoreType.DMA,
        ],
    )
    def k(t_hbm, p_hbm, row_hbm, col_hbm, gr_hbm, gc_hbm,
          rbuf, cbuf, tbuf0, tbuf1, pbuf0, pbuf1, gsem0, gsem1, wsem0, wsem1):
        c = lax.axis_index("c")
        s = lax.axis_index("s")
        wid = s * NC + c
        pltpu.sync_copy(row_hbm.at[wid], rbuf)
        pltpu.sync_copy(col_hbm.at[wid], cbuf)

        def gath(j, tb, pb, gsem):
            pltpu.async_copy(t_hbm.at[rbuf.at[j]], tb, gsem)
            pltpu.async_copy(p_hbm.at[cbuf.at[j]], pb, gsem)

        def wait_g(tb, pb, gsem):
            pltpu.make_async_copy(t_hbm.at[rbuf.at[0]], tb, gsem).wait()
            pltpu.make_async_copy(p_hbm.at[cbuf.at[0]], pb, gsem).wait()

        def wait_w(tb, pb, wsem):
            pltpu.make_async_copy(tb, gr_hbm.at[pl.ds(0, CH)], wsem).wait()
            pltpu.make_async_copy(pb, gc_hbm.at[pl.ds(0, CH)], wsem).wait()

        def do_iter(j, tb, pb, gsem, wsem, tbo, pbo, gsemo, wsemo):
            @pl.when(j + 1 < NCH)
            def _():
                @pl.when(j >= 1)
                def _():
                    wait_w(tbo, pbo, wsemo)
                gath(j + 1, tbo, pbo, gsemo)
            wait_g(tb, pb, gsem)
            e0 = (wid * NCH + j) * CH
            pltpu.async_copy(tb, gr_hbm.at[pl.ds(e0, CH)], wsem)
            pltpu.async_copy(pb, gc_hbm.at[pl.ds(e0, CH)], wsem)

        gath(0, tbuf0, pbuf0, gsem0)

        def body(j, carry):
            @pl.when(j % 2 == 0)
            def _():
                do_iter(j, tbuf0, pbuf0, gsem0, wsem0,
                        tbuf1, pbuf1, gsem1, wsem1)

            @pl.when(j % 2 == 1)
            def _():
                do_iter(j, tbuf1, pbuf1, gsem1, wsem1,
                        tbuf0, pbuf0, gsem0, wsem0)
            return carry

        lax.fori_loop(0, NCH, body, 0)
        wait_w(tbuf0, pbuf0, wsem0)
        wait_w(tbuf1, pbuf1, wsem1)

    return k


@functools.cache
def _build_sc_scatter(ds_, read_vals, n_e, CH):
    NCH = n_e // NW // CH
    scratch = [
        pltpu.VMEM((NCH, CH), jnp.int32),
        pltpu.VMEM((CH, ds_), jnp.float32),
        pltpu.VMEM((CH, ds_), jnp.float32),
        pltpu.VMEM_SHARED((N, ds_), jnp.float32),
        pltpu.SemaphoreType.DMA,
        pltpu.SemaphoreType.DMA,
    ]

    def body(vals_hbm, col_hbm, zeros_hbm, out_hbm, cbuf, vbuf0, vbuf1, acc,
             lsem0, lsem1):
        c = lax.axis_index("c")
        s = lax.axis_index("s")
        wid = s * NC + c

        @pl.when(s == 0)
        def _():
            pltpu.sync_copy(zeros_hbm, acc)

        if not read_vals:
            def fill(i, carry):
                vbuf0[i] = jnp.full((ds_,), 1.0, jnp.float32)
                return carry
            lax.fori_loop(0, CH, fill, 0)

        plsc.subcore_barrier()
        pltpu.sync_copy(col_hbm.at[wid], cbuf)

        if read_vals:
            def load(j, vb, lsem):
                e0 = (wid * NCH + j) * CH
                pltpu.async_copy(vals_hbm.at[pl.ds(e0, CH)], vb, lsem)

            def wait_l(vb, lsem):
                pltpu.make_async_copy(vals_hbm.at[pl.ds(0, CH)], vb, lsem
                                      ).wait()

            def do_iter(j, vb, lsem, vbo, lsemo):
                @pl.when(j + 1 < NCH)
                def _():
                    load(j + 1, vbo, lsemo)
                wait_l(vb, lsem)
                pltpu.sync_copy(vb, acc.at[cbuf.at[j]], add=True)

            load(0, vbuf0, lsem0)

            def chunk(j, carry):
                @pl.when(j % 2 == 0)
                def _():
                    do_iter(j, vbuf0, lsem0, vbuf1, lsem1)

                @pl.when(j % 2 == 1)
                def _():
                    do_iter(j, vbuf1, lsem1, vbuf0, lsem0)
                return carry
        else:
            def chunk(j, carry):
                pltpu.sync_copy(vbuf0, acc.at[cbuf.at[j]], add=True)
                return carry

        lax.fori_loop(0, NCH, chunk, 0)
        plsc.subcore_barrier()

        @pl.when(s == 0)
        def _():
            pltpu.sync_copy(acc, out_hbm.at[c])

    if read_vals:
        def k(vals_hbm, col_hbm, zeros_hbm, out_hbm, cbuf, vbuf0, vbuf1, acc,
              lsem0, lsem1):
            body(vals_hbm, col_hbm, zeros_hbm, out_hbm, cbuf, vbuf0, vbuf1,
                 acc, lsem0, lsem1)
    else:
        def k(col_hbm, zeros_hbm, out_hbm, cbuf, vbuf0, vbuf1, acc, lsem0,
              lsem1):
            body(None, col_hbm, zeros_hbm, out_hbm, cbuf, vbuf0, vbuf1, acc,
                 lsem0, lsem1)

    return pl.kernel(
        k,
        out_type=jax.ShapeDtypeStruct((NC, N, ds_), jnp.float32),
        mesh=_mesh(),
        scratch_types=scratch,
    )


def _dot(a, b):
    return jnp.dot(a, b, preferred_element_type=jnp.float32)


@functools.cache
def _build_prep(dea):
    def body(x, u, b2, we1, wn1, be1, bn1, t_out, p_out):
        onehot = (b2[...] == lax.broadcasted_iota(jnp.int32, (1, NG), 1)
                  ).astype(jnp.float32)
        uw = _dot(u[...], we1[2 * D + dea:3 * D + dea, :])
        q = _dot(x[...], we1[0:D, :]) + _dot(onehot, uw) + be1[...]
        pn = _dot(x[...], wn1[0:D, :]) + bn1[...]
        t_out[...] = jnp.concatenate([q, pn], axis=1)
        p_out[...] = _dot(x[...], we1[D:2 * D, :])

    return pl.pallas_call(
        body,
        out_shape=(jax.ShapeDtypeStruct((N, 2 * D), jnp.float32),
                   jax.ShapeDtypeStruct((N, D), jnp.float32)),
    )


def _affine_step0(st_refs, g_ref, bb_ref, scr_ref):
    @pl.when(pl.program_id(0) == 0)
    def _():
        ssum = sum(jnp.sum(r[...], axis=0) for r in st_refs)
        mean = ssum[0:1] * (1.0 / E)
        var = ssum[1:2] * (1.0 / E) - mean * mean
        a = g_ref[...] * lax.rsqrt(var + 1e-5)
        scr_ref[...] = jnp.concatenate([a, bb_ref[...] - mean * a], axis=0)


def _pre_apply(x, scr_ref):
    return _selu(x * scr_ref[0:1] + scr_ref[1:2])


@functools.cache
def _build_pass(d_x, w_row0, pst_nblks, adds_cols, stats, n_e, nblk):
    npst = len(pst_nblks)
    nadds = len(adds_cols)

    def body(*refs):
        i = 0
        x_ref = refs[i]; i += 1
        w_ref = refs[i]; i += 1
        b_ref = refs[i]; i += 1
        pst_refs = refs[i:i + npst]; i += npst
        if npst:
            g_ref = refs[i]; i += 1
            bb_ref = refs[i]; i += 1
        add_refs = refs[i:i + nadds]; i += nadds
        z_ref = refs[i]; i += 1
        st_ref = refs[i] if stats else None
        scr_ref = refs[-1]

        xb = x_ref[...]
        if npst:
            _affine_step0(pst_refs, g_ref, bb_ref, scr_ref)
            xb = _pre_apply(xb, scr_ref)
        z = _dot(xb, w_ref[w_row0:w_row0 + d_x, :]) + b_ref[...]
        for ad in add_refs:
            z = z + ad[...]
        z_ref[...] = z
        if stats:
            st_ref[...] = jnp.stack(
                [jnp.sum(z, axis=0), jnp.sum(z * z, axis=0)])[None]

    def wspec(nrows):
        return pl.BlockSpec((nrows, D), lambda i: (0, 0))

    in_specs = [
        pl.BlockSpec((BE, d_x), lambda i: (i, 0)),
        wspec(w_row0 + d_x),
        wspec(1),
    ]
    in_specs += [pl.BlockSpec((nbk, 2, D), lambda i: (0, 0, 0))
                 for nbk in pst_nblks]
    if npst:
        in_specs += [wspec(1), wspec(1)]
    for cb in adds_cols:
        in_specs.append(pl.BlockSpec((BE, D), lambda i, cb=cb: (i, cb)))
    out_shape = [jax.ShapeDtypeStruct((n_e, D), jnp.float32)]
    out_specs = [pl.BlockSpec((BE, D), lambda i: (i, 0))]
    if stats:
        out_shape.append(jax.ShapeDtypeStruct((nblk, 2, D), jnp.float32))
        out_specs.append(pl.BlockSpec((1, 2, D), lambda i: (i, 0, 0)))

    return pl.pallas_call(
        body,
        grid=(n_e // BE,),
        in_specs=in_specs,
        out_specs=out_specs,
        out_shape=out_shape,
        scratch_shapes=[pltpu.VMEM((2, D), jnp.float32)],
    )


@functools.cache
def _build_pass_e3(pst_nblks, n_e, nblk):
    npst = len(pst_nblks)
    def body(*refs):
        i = 0
        z2 = refs[i]; i += 1
        pst_refs = refs[i:i + npst]; i += npst
        g2, bb2, w3, b3, wn, g = refs[i:i + 6]; i += 6
        eo_ref, nz_ref, st_ref, scr = refs[i:i + 4]
        _affine_step0(pst_refs, g2, bb2, scr)
        a2 = _pre_apply(z2[...], scr)
        eo = _dot(a2, w3[...]) + b3[...]
        eo_ref[...] = eo
        nz = _dot(eo, wn[D:2 * D, :]) + g[...]
        nz_ref[...] = nz
        st_ref[...] = jnp.stack(
            [jnp.sum(nz, axis=0), jnp.sum(nz * nz, axis=0)])[None]

    return pl.pallas_call(
        body,
        grid=(n_e // BE,),
        in_specs=[
            pl.BlockSpec((BE, D), lambda i: (i, 0)),
        ] + [pl.BlockSpec((nbk, 2, D), lambda i: (0, 0, 0))
             for nbk in pst_nblks] + [
            pl.BlockSpec((1, D), lambda i: (0, 0)),
            pl.BlockSpec((1, D), lambda i: (0, 0)),
            pl.BlockSpec((D, D), lambda i: (0, 0)),
            pl.BlockSpec((1, D), lambda i: (0, 0)),
            pl.BlockSpec((2 * D, D), lambda i: (0, 0)),
            pl.BlockSpec((BE, D), lambda i: (i, 1)),
        ],
        out_specs=[
            pl.BlockSpec((BE, D), lambda i: (i, 0)),
            pl.BlockSpec((BE, D), lambda i: (i, 0)),
            pl.BlockSpec((1, 2, D), lambda i: (i, 0, 0)),
        ],
        out_shape=[
            jax.ShapeDtypeStruct((n_e, D), jnp.float32),
            jax.ShapeDtypeStruct((n_e, D), jnp.float32),
            jax.ShapeDtypeStruct((nblk, 2, D), jnp.float32),
        ],
        scratch_shapes=[pltpu.VMEM((2, D), jnp.float32)],
    )


def _bn(z, g, bb):
    mean = jnp.mean(z, axis=0, keepdims=True)
    var = jnp.var(z, axis=0, keepdims=True)
    return (z - mean) * lax.rsqrt(var + 1e-5) * g + bb


@functools.cache
def _build_node_fused(nseg):
    def body(*refs):
        x = refs[0]
        accs = refs[1:1 + nseg]
        (cinv, u, b2,
         wm1, bm1, gm1, gmb1, wm2, bm2, gm2, gmb2, wm3, bm3,
         wg1, bg1, gg1, ggb1, wg2, bg2, gg2, ggb2, wg3, bg3,
         xn_ref, un_ref) = refs[1 + nseg:]
        agg = sum(a[0] + a[1] for a in accs) * cinv[...]
        onehot = (b2[...] == lax.broadcasted_iota(jnp.int32, (1, NG), 1)
                  ).astype(jnp.float32)
        ub = _dot(onehot, _dot(u[...], wm1[2 * D:3 * D, :]))
        z = (_dot(x[...], wm1[0:D, :]) + _dot(agg, wm1[D:2 * D, :])
             + ub + bm1[...])
        h1 = _selu(_bn(z, gm1[...], gmb1[...]))
        z2 = _dot(h1, wm2[...]) + bm2[...]
        h2 = _selu(_bn(z2, gm2[...], gmb2[...]))
        xn = _dot(h2, wm3[...]) + bm3[...]
        xn_ref[...] = xn

        nc = jnp.sum(onehot, axis=0)
        nmean = lax.dot_general(onehot, xn, (((0,), (0,)), ((), ()))
                                ) / jnp.maximum(nc, 1.0)[:, None]
        gz = (_dot(u[...], wg1[0:D, :]) + _dot(nmean, wg1[D:2 * D, :])
              + bg1[...])
        d1 = _selu(_bn(gz, gg1[...], ggb1[...]))
        gz2 = _dot(d1, wg2[...]) + bg2[...]
        d2 = _selu(_bn(gz2, gg2[...], ggb2[...]))
        un_ref[...] = _dot(d2, wg3[...]) + bg3[...]

    return pl.pallas_call(
        body,
        out_shape=(jax.ShapeDtypeStruct((N, D), jnp.float32),
                   jax.ShapeDtypeStruct((NG, D), jnp.float32)),
    )


def _r(v):
    return v.reshape(1, D)


SPLITS = ((0, 192000), (192000, 128000))


def kernel(x, edge_attr, u, params, edge_index, batch):
    row = edge_index[0]
    col = edge_index[1]
    col2_full = col.reshape(NW, E // NW // CH, CH)
    row2 = [row[s:s + n].reshape(NW, n // NW // CH, CH) for s, n in SPLITS]
    col2 = [col[s:s + n].reshape(NW, n // NW // CH, CH) for s, n in SPLITS]
    batch2 = batch.reshape(N, 1)
    zeros128 = jnp.zeros((N, D), jnp.float32)

    counts = _build_sc_scatter(D, False, E, CH)(col2_full, zeros128)
    c = counts[0, :, 0] + counts[1, :, 0]
    cinv = (1.0 / jnp.maximum(c, 1.0)).reshape(N, 1)

    nseg = len(SPLITS)
    ea = [edge_attr[s:s + n] if nseg > 1 else edge_attr for s, n in SPLITS]
    H = range(nseg)
    for lp in params:
        (We1, be1, g1, bb1), (We2, be2, g2, bb2), (We3, be3) = lp['edge']
        (Wn1, bn1, gn1, bnb1), (Wn2, bn2, gn2, bnb2), (Wn3, bn3) = lp['node1']
        (Wm1, bm1, gm1, gmb1), (Wm2, bm2, gm2, gmb2), (Wm3, bm3) = lp['node2']
        (Wg1, bg1, gg1, ggb1), (Wg2, bg2, gg2, ggb2), (Wg3, bg3) = lp['global']
        dea = We1.shape[0] - (2 * D + D)

        T, P = _build_prep(dea)(
            x, u, batch2, We1, Wn1, _r(be1), _r(bn1))
        G = [_build_sc_gather(SPLITS[h][1], CH)(T, P, row2[h], col2[h])
             for h in H]

        zb = jnp.zeros((1, D), jnp.float32)

        def nb(h):
            return SPLITS[h][1] // BE

        nbs = tuple(nb(h) for h in H)
        e1 = [_build_pass(dea, 2 * D, (), (0, 0), True, SPLITS[h][1], nb(h))(
            ea[h], We1, zb, G[h][0], G[h][1]) for h in H]
        st1 = [e1[h][1] for h in H]
        e2 = [_build_pass(D, 0, nbs, (), True, SPLITS[h][1], nb(h))(
            e1[h][0], We2, _r(be2), *st1, _r(g1), _r(bb1))
            for h in H]
        st2 = [e2[h][1] for h in H]
        e3 = [_build_pass_e3(nbs, SPLITS[h][1], nb(h))(
            e2[h][0], *st2, _r(g2), _r(bb2), We3, _r(be3), Wn1,
            G[h][0]) for h in H]
        st3 = [e3[h][2] for h in H]
        n2 = [_build_pass(D, 0, nbs, (), True, SPLITS[h][1], nb(h))(
            e3[h][1], Wn2, _r(bn2), *st3, _r(gn1), _r(bnb1))
            for h in H]
        st4 = [n2[h][1] for h in H]
        hh = [_build_pass(D, 0, nbs, (), False, SPLITS[h][1], nb(h))(
            n2[h][0], Wn3, _r(bn3), *st4, _r(gn2), _r(bnb2))[0]
            for h in H]

        acc = [_build_sc_scatter(D, True, SPLITS[h][1], CH)(
            hh[h], col2[h], zeros128) for h in H]

        x, u = _build_node_fused(nseg)(
            x, *acc, cinv, u, batch2,
            Wm1, _r(bm1), _r(gm1), _r(gmb1),
            Wm2, _r(bm2), _r(gm2), _r(gmb2), Wm3, _r(bm3),
            Wg1, _r(bg1), _r(gg1), _r(ggb1),
            Wg2, _r(bg2), _r(gg2), _r(ggb2), Wg3, _r(bg3))
        ea = [e3[h][0] for h in H]
    return u
